# Initial kernel scaffold; baseline (speedup 1.0000x reference)
import jax
import jax.numpy as jnp
from jax.experimental import pallas as pl


def kernel(x, edge_index, Wp, bp, W0, b0, W1, b1, Wt, bt, Wih2, Whh2, bih2, bhh2, Wih3, Whh3, bih3, bhh3, Wout, bout):
    raise NotImplementedError("write your pallas kernel here")


# speedup vs baseline: 2.2152x; 2.2152x over previous
import functools

import jax
import jax.numpy as jnp
from jax import lax
from jax.experimental import pallas as pl
from jax.experimental.pallas import tpu as pltpu
from jax.experimental.pallas import tpu_sc as plsc

_N = 10000
_E = 320000
_H = 128
_MAXR = 10
_TAU = 0.005

_NW = 32
_EPW = _E // _NW
_K = 80
_NCH = _EPW // _K
_RCH = _N // _K

_BP = 2000
_BG = 400
_NBG = _N // _BG


def _edge_agg(h, row, col):
    mesh = plsc.VectorSubcoreMesh(
        core_axis_name="c", subcore_axis_name="s", num_cores=2, num_subcores=16)

    @functools.partial(
        pl.kernel,
        out_type=jax.ShapeDtypeStruct((2, _N, _H), jnp.float32),
        mesh=mesh,
        scratch_types=[
            pltpu.VMEM((_K,), jnp.int32),
            pltpu.VMEM((_K,), jnp.int32),
            pltpu.VMEM((_K, _H), jnp.float32),
            pltpu.VMEM((_K, _H), jnp.float32),
            pltpu.VMEM((_K, _H), jnp.float32),
            pltpu.VMEM_SHARED((_N, _H), jnp.float32),
            pltpu.SemaphoreType.DMA,
            pltpu.SemaphoreType.DMA,
        ],
    )
    def ek(h_hbm, row_hbm, col_hbm, out_hbm,
           idx_r, idx_c, buf_a, buf_b, buf_d, acc, sem1, sem2):
        c = lax.axis_index("c")
        s = lax.axis_index("s")
        w = c * 16 + s

        def zero_body(i, _):
            for q in range(_H // 16):
                buf_d[i, pl.ds(q * 16, 16)] = jnp.zeros((16,), jnp.float32)
            return 0
        lax.fori_loop(0, _K, zero_body, 0)

        for t in range(8):
            j = s + t * 16

            @pl.when(j < _RCH)
            def _():
                pltpu.sync_copy(buf_d, acc.at[pl.ds(j * _K, _K)])

        plsc.subcore_barrier()

        def chunk_body(ch, _):
            base = w * _EPW + ch * _K
            pltpu.sync_copy(row_hbm.at[pl.ds(base, _K)], idx_r)
            pltpu.sync_copy(col_hbm.at[pl.ds(base, _K)], idx_c)
            cp1 = pltpu.async_copy(h_hbm.at[idx_r], buf_a, sem1)
            cp2 = pltpu.async_copy(h_hbm.at[idx_c], buf_b, sem2)
            cp1.wait()
            cp2.wait()

            def diff_body(i, _):
                for q in range(_H // 16):
                    sl = pl.ds(q * 16, 16)
                    buf_d[i, sl] = jnp.abs(buf_a[i, sl] - buf_b[i, sl])
                return 0
            lax.fori_loop(0, _K, diff_body, 0)

            pltpu.sync_copy(buf_d, acc.at[idx_r], add=True)
            return 0
        lax.fori_loop(0, _NCH, chunk_body, 0)

        plsc.subcore_barrier()

        for t in range(8):
            j = s + t * 16

            @pl.when(j < _RCH)
            def _():
                pltpu.sync_copy(acc.at[pl.ds(j * _K, _K)], buf_d)
                pltpu.sync_copy(buf_d, out_hbm.at[c, pl.ds(j * _K, _K)])

    return ek(h, row, col)


def _proj(x, wt, b):
    def body(x_ref, w_ref, b_ref, o_ref):
        o_ref[...] = jax.nn.relu(
            jnp.dot(x_ref[...], w_ref[...], preferred_element_type=jnp.float32)
            + b_ref[...])

    return pl.pallas_call(
        body,
        grid=(_N // _BP,),
        in_specs=[
            pl.BlockSpec((_BP, _H), lambda i: (i, 0)),
            pl.BlockSpec((_H, _H), lambda i: (0, 0)),
            pl.BlockSpec((1, _H), lambda i: (0, 0)),
        ],
        out_specs=pl.BlockSpec((_BP, _H), lambda i: (i, 0)),
        out_shape=jax.ShapeDtypeStruct((_N, _H), jnp.float32),
    )(x, wt, b)


def _dense_layer(h, agg0, agg1, wat, wbt, b, wtt, bt):
    def body(h_ref, a0_ref, a1_ref, wa_ref, wb_ref, b_ref, wt_ref, bt_ref,
             h2_ref, tau_ref):
        agg = a0_ref[...] + a1_ref[...]
        h2 = jax.nn.relu(
            jnp.dot(h_ref[...], wa_ref[...], preferred_element_type=jnp.float32)
            + jnp.dot(agg, wb_ref[...], preferred_element_type=jnp.float32)
            + b_ref[...])
        h2_ref[...] = h2
        t = jnp.dot(h2, wt_ref[...], preferred_element_type=jnp.float32)
        tau_ref[...] = jax.nn.softplus(t + bt_ref[...])

    return pl.pallas_call(
        body,
        grid=(_N // _BP,),
        in_specs=[
            pl.BlockSpec((_BP, _H), lambda i: (i, 0)),
            pl.BlockSpec((_BP, _H), lambda i: (i, 0)),
            pl.BlockSpec((_BP, _H), lambda i: (i, 0)),
            pl.BlockSpec((_H, _H), lambda i: (0, 0)),
            pl.BlockSpec((_H, _H), lambda i: (0, 0)),
            pl.BlockSpec((1, _H), lambda i: (0, 0)),
            pl.BlockSpec((_H, 1), lambda i: (0, 0)),
            pl.BlockSpec((1, 1), lambda i: (0, 0)),
        ],
        out_specs=[
            pl.BlockSpec((_BP, _H), lambda i: (i, 0)),
            pl.BlockSpec((_BP, 1), lambda i: (i, 0)),
        ],
        out_shape=[
            jax.ShapeDtypeStruct((_N, _H), jnp.float32),
            jax.ShapeDtypeStruct((_N, 1), jnp.float32),
        ],
    )(h, agg0, agg1, wat, wbt, b, wtt, bt)


def _gru_recursion(hm, count, a2, n2, brz2, bn2, a3, n3, brz3, bn3):
    def cell(hv, a_ref, brz_ref, n_ref, bn_ref):
        grz = jnp.dot(hv, a_ref[...], preferred_element_type=jnp.float32) + brz_ref[...]
        r = jax.nn.sigmoid(grz[:, :_H])
        z = jax.nn.sigmoid(grz[:, _H:])
        gn = jnp.dot(hv, n_ref[...], preferred_element_type=jnp.float32) + bn_ref[...]
        n = jnp.tanh(gn[:, :_H] + r * gn[:, _H:])
        return (1.0 - z) * n + z * hv

    def body(cnt_ref, hm_ref, a2_ref, brz2_ref, n2_ref, bn2_ref,
             a3_ref, brz3_ref, n3_ref, bn3_ref, o_ref):
        pid = pl.program_id(0)

        @pl.when(pid * _BG < cnt_ref[0])
        def _():
            hv0 = hm_ref[...]

            def it(_, hv):
                hv = cell(hv, a2_ref, brz2_ref, n2_ref, bn2_ref)
                return cell(hv, a3_ref, brz3_ref, n3_ref, bn3_ref)

            o_ref[...] = lax.fori_loop(0, _MAXR, it, hv0)

    wspec = lambda sh: pl.BlockSpec(sh, lambda i, cnt: (0, 0))
    grid_spec = pltpu.PrefetchScalarGridSpec(
        num_scalar_prefetch=1,
        grid=(_NBG,),
        in_specs=[
            pl.BlockSpec((_BG, _H), lambda i, cnt: (i, 0)),
            wspec((_H, 2 * _H)), wspec((1, 2 * _H)),
            wspec((_H, 2 * _H)), wspec((1, 2 * _H)),
            wspec((_H, 2 * _H)), wspec((1, 2 * _H)),
            wspec((_H, 2 * _H)), wspec((1, 2 * _H)),
        ],
        out_specs=pl.BlockSpec((_BG, _H), lambda i, cnt: (i, 0)),
    )
    return pl.pallas_call(
        body,
        grid_spec=grid_spec,
        out_shape=jax.ShapeDtypeStruct((_N, _H), jnp.float32),
    )(count, hm, a2, brz2, n2, bn2, a3, brz3, n3, bn3)


def _final(h, resg, tau, wt, b):
    def body(h_ref, r_ref, tau_ref, w_ref, b_ref, o_ref):
        hf = jnp.where(tau_ref[...] < _TAU, r_ref[...], h_ref[...])
        o_ref[...] = (
            jnp.dot(hf, w_ref[...], preferred_element_type=jnp.float32)
            + b_ref[...])

    return pl.pallas_call(
        body,
        grid=(_N // _BP,),
        in_specs=[
            pl.BlockSpec((_BP, _H), lambda i: (i, 0)),
            pl.BlockSpec((_BP, _H), lambda i: (i, 0)),
            pl.BlockSpec((_BP, 1), lambda i: (i, 0)),
            pl.BlockSpec((_H, 2), lambda i: (0, 0)),
            pl.BlockSpec((1, 2), lambda i: (0, 0)),
        ],
        out_specs=pl.BlockSpec((_BP, 2), lambda i: (i, 0)),
        out_shape=jax.ShapeDtypeStruct((_N, 2), jnp.float32),
    )(h, resg, tau, wt, b)


def kernel(x, edge_index, Wp, bp, W0, b0, W1, b1, Wt, bt,
           Wih2, Whh2, bih2, bhh2, Wih3, Whh3, bih3, bhh3, Wout, bout):
    row = edge_index[0]
    col = edge_index[1]

    wpt = Wp.T
    bp2 = bp[None, :]
    w0t = W0.T
    w1t = W1.T
    wtt = Wt.T
    bt2 = bt[None, :]
    a2 = (Wih2[:2 * _H] + Whh2[:2 * _H]).T
    brz2 = (bih2[:2 * _H] + bhh2[:2 * _H])[None, :]
    n2 = jnp.concatenate([Wih2[2 * _H:], Whh2[2 * _H:]], axis=0).T
    bn2 = jnp.concatenate([bih2[2 * _H:], bhh2[2 * _H:]])[None, :]
    a3 = (Wih3[:2 * _H] + Whh3[:2 * _H]).T
    brz3 = (bih3[:2 * _H] + bhh3[:2 * _H])[None, :]
    n3 = jnp.concatenate([Wih3[2 * _H:], Whh3[2 * _H:]], axis=0).T
    bn3 = jnp.concatenate([bih3[2 * _H:], bhh3[2 * _H:]])[None, :]
    woutt = Wout.T
    bout2 = bout[None, :]

    h = _proj(x, wpt, bp2)

    def layer(h, wlt, bl):
        parts = _edge_agg(h, row, col)
        h2, tau = _dense_layer(h, parts[0], parts[1],
                               wlt[:_H], wlt[_H:], bl[None, :], wtt, bt2)
        mask = tau[:, 0] < _TAU
        count = jnp.sum(mask.astype(jnp.int32))[None]
        idx = jnp.nonzero(mask, size=_N, fill_value=0)[0].astype(jnp.int32)
        hm = h2[idx]
        res = _gru_recursion(hm, count, a2, n2, brz2, bn2, a3, n3, brz3, bn3)
        pos = jnp.cumsum(mask.astype(jnp.int32)) - 1
        resg = res[jnp.where(mask, pos, 0)]
        return h2, resg, tau, mask

    h1, resg1, tau1, mask1 = layer(h, w0t, b0)
    h1f = jnp.where(mask1[:, None], resg1, h1)
    h2, resg2, tau2, _ = layer(h1f, w1t, b1)
    return _final(h2, resg2, tau2, woutt, bout2)


# speedup vs baseline: 2.5241x; 1.1394x over previous
import functools

import jax
import jax.numpy as jnp
from jax import lax
from jax.experimental import pallas as pl
from jax.experimental.pallas import tpu as pltpu
from jax.experimental.pallas import tpu_sc as plsc

_N = 10000
_E = 320000
_H = 128
_MAXR = 10
_TAU = 0.005

_NW = 32
_EPW = _E // _NW
_K = 80
_NCH = _EPW // _K
_RCH = _N // _K

_BP = 2000
_BG = 400
_NBG = _N // _BG


def _sc_mesh():
    return plsc.VectorSubcoreMesh(
        core_axis_name="c", subcore_axis_name="s", num_cores=2, num_subcores=16)


def _edge_agg(h, row, col):
    @functools.partial(
        pl.kernel,
        out_type=jax.ShapeDtypeStruct((2, _N, _H), jnp.float32),
        mesh=_sc_mesh(),
        scratch_types=[
            pltpu.VMEM((_K,), jnp.int32),
            pltpu.VMEM((_K,), jnp.int32),
            pltpu.VMEM((_K,), jnp.int32),
            pltpu.VMEM((_K,), jnp.int32),
            pltpu.VMEM((_K, _H), jnp.float32),
            pltpu.VMEM((_K, _H), jnp.float32),
            pltpu.VMEM((_K, _H), jnp.float32),
            pltpu.VMEM((_K, _H), jnp.float32),
            pltpu.VMEM_SHARED((_N, _H), jnp.float32),
            pltpu.SemaphoreType.DMA,
            pltpu.SemaphoreType.DMA,
            pltpu.SemaphoreType.DMA,
            pltpu.SemaphoreType.DMA,
        ],
    )
    def ek(h_hbm, row_hbm, col_hbm, out_hbm,
           idx_r0, idx_c0, idx_r1, idx_c1,
           buf_a0, buf_b0, buf_a1, buf_b1, acc,
           sem_a0, sem_b0, sem_a1, sem_b1):
        c = lax.axis_index("c")
        s = lax.axis_index("s")
        w = c * 16 + s

        def zero_body(i, _):
            for q in range(_H // 16):
                buf_a0[i, pl.ds(q * 16, 16)] = jnp.zeros((16,), jnp.float32)
            return 0
        lax.fori_loop(0, _K, zero_body, 0)

        for t in range(8):
            j = s + t * 16

            @pl.when(j < _RCH)
            def _():
                pltpu.sync_copy(buf_a0, acc.at[pl.ds(j * _K, _K)])

        plsc.subcore_barrier()

        def issue(ch, idx_r, idx_c, buf_a, buf_b, sem_a, sem_b):
            base = w * _EPW + ch * _K
            pltpu.sync_copy(row_hbm.at[pl.ds(base, _K)], idx_r)
            pltpu.sync_copy(col_hbm.at[pl.ds(base, _K)], idx_c)
            pltpu.async_copy(h_hbm.at[idx_r], buf_a, sem_a)
            pltpu.async_copy(h_hbm.at[idx_c], buf_b, sem_b)

        def consume(idx_r, idx_c, buf_a, buf_b, sem_a, sem_b):
            pltpu.make_async_copy(h_hbm.at[idx_r], buf_a, sem_a).wait()
            pltpu.make_async_copy(h_hbm.at[idx_c], buf_b, sem_b).wait()

            def diff_body(i, _):
                for q in range(_H // 16):
                    sl = pl.ds(q * 16, 16)
                    buf_a[i, sl] = jnp.abs(buf_a[i, sl] - buf_b[i, sl])
                return 0
            lax.fori_loop(0, _K, diff_body, 0)
            pltpu.sync_copy(buf_a, acc.at[idx_r], add=True)

        issue(0, idx_r0, idx_c0, buf_a0, buf_b0, sem_a0, sem_b0)

        def pair_body(t, _):
            ch0 = 2 * t
            issue(ch0 + 1, idx_r1, idx_c1, buf_a1, buf_b1, sem_a1, sem_b1)
            consume(idx_r0, idx_c0, buf_a0, buf_b0, sem_a0, sem_b0)
            issue(ch0 + 2, idx_r0, idx_c0, buf_a0, buf_b0, sem_a0, sem_b0)
            consume(idx_r1, idx_c1, buf_a1, buf_b1, sem_a1, sem_b1)
            return 0
        lax.fori_loop(0, (_NCH - 1) // 2, pair_body, 0)

        consume(idx_r0, idx_c0, buf_a0, buf_b0, sem_a0, sem_b0)

        plsc.subcore_barrier()

        for t in range(8):
            j = s + t * 16

            @pl.when(j < _RCH)
            def _():
                pltpu.sync_copy(acc.at[pl.ds(j * _K, _K)], buf_a0)
                pltpu.sync_copy(buf_a0, out_hbm.at[c, pl.ds(j * _K, _K)])

    return ek(h, row, col)


def _compact_gather(h2, idxg):
    @functools.partial(
        pl.kernel,
        out_type=jax.ShapeDtypeStruct((_N, _H), jnp.float32),
        mesh=_sc_mesh(),
        scratch_types=[
            pltpu.VMEM((_K,), jnp.int32),
            pltpu.VMEM((_K, _H), jnp.float32),
            pltpu.SemaphoreType.DMA,
        ],
    )
    def gk(h_hbm, idx_hbm, out_hbm, idx_v, buf, sem):
        c = lax.axis_index("c")
        s = lax.axis_index("s")
        w = c * 16 + s
        for t in range(4):
            j = w + t * _NW

            @pl.when(j < _RCH)
            def _():
                pltpu.sync_copy(idx_hbm.at[pl.ds(j * _K, _K)], idx_v)
                pltpu.async_copy(h_hbm.at[idx_v], buf, sem).wait()
                pltpu.sync_copy(buf, out_hbm.at[pl.ds(j * _K, _K)])

    return gk(h2, idxg)


def _merge_scatter(h2, res, idxs):
    @functools.partial(
        pl.kernel,
        out_type=jax.ShapeDtypeStruct((_N + 8, _H), jnp.float32),
        mesh=_sc_mesh(),
        scratch_types=[
            pltpu.VMEM((_K,), jnp.int32),
            pltpu.VMEM((_K, _H), jnp.float32),
            pltpu.SemaphoreType.DMA,
        ],
    )
    def mk(h_hbm, res_hbm, idx_hbm, out_hbm, idx_v, buf, sem):
        c = lax.axis_index("c")
        s = lax.axis_index("s")

        @pl.when(c == 0)
        def _():
            for t in range(8):
                j = s + t * 16

                @pl.when(j < _RCH)
                def _():
                    pltpu.sync_copy(h_hbm.at[pl.ds(j * _K, _K)], buf)
                    pltpu.sync_copy(buf, out_hbm.at[pl.ds(j * _K, _K)])

            plsc.subcore_barrier()

            for t in range(8):
                j = s + t * 16

                @pl.when(j < _RCH)
                def _():
                    pltpu.sync_copy(idx_hbm.at[pl.ds(j * _K, _K)], idx_v)
                    pltpu.sync_copy(res_hbm.at[pl.ds(j * _K, _K)], buf)
                    pltpu.sync_copy(buf, out_hbm.at[idx_v])

    return mk(h2, res, idxs)


def _proj(x, wt, b):
    def body(x_ref, w_ref, b_ref, o_ref):
        o_ref[...] = jax.nn.relu(
            jnp.dot(x_ref[...], w_ref[...], preferred_element_type=jnp.float32)
            + b_ref[...])

    return pl.pallas_call(
        body,
        grid=(_N // _BP,),
        in_specs=[
            pl.BlockSpec((_BP, _H), lambda i: (i, 0)),
            pl.BlockSpec((_H, _H), lambda i: (0, 0)),
            pl.BlockSpec((1, _H), lambda i: (0, 0)),
        ],
        out_specs=pl.BlockSpec((_BP, _H), lambda i: (i, 0)),
        out_shape=jax.ShapeDtypeStruct((_N, _H), jnp.float32),
    )(x, wt, b)


def _dense_layer(h, agg0, agg1, wlt, b, wtt, bt):
    def body(h_ref, a0_ref, a1_ref, wl_ref, b_ref, wt_ref, bt_ref,
             h2_ref, tau_ref):
        hc = jnp.concatenate([h_ref[...], a0_ref[...] + a1_ref[...]], axis=-1)
        h2 = jax.nn.relu(
            jnp.dot(hc, wl_ref[...], preferred_element_type=jnp.float32)
            + b_ref[...])
        h2_ref[...] = h2
        t = jnp.dot(h2, wt_ref[...], preferred_element_type=jnp.float32)
        tau_ref[...] = jax.nn.softplus(t + bt_ref[...])

    return pl.pallas_call(
        body,
        grid=(_N // _BP,),
        in_specs=[
            pl.BlockSpec((_BP, _H), lambda i: (i, 0)),
            pl.BlockSpec((_BP, _H), lambda i: (i, 0)),
            pl.BlockSpec((_BP, _H), lambda i: (i, 0)),
            pl.BlockSpec((2 * _H, _H), lambda i: (0, 0)),
            pl.BlockSpec((1, _H), lambda i: (0, 0)),
            pl.BlockSpec((_H, 1), lambda i: (0, 0)),
            pl.BlockSpec((1, 1), lambda i: (0, 0)),
        ],
        out_specs=[
            pl.BlockSpec((_BP, _H), lambda i: (i, 0)),
            pl.BlockSpec((_BP, 1), lambda i: (i, 0)),
        ],
        out_shape=[
            jax.ShapeDtypeStruct((_N, _H), jnp.float32),
            jax.ShapeDtypeStruct((_N, 1), jnp.float32),
        ],
    )(h, agg0, agg1, wlt, b, wtt, bt)


def _gru_recursion(hm, count, wih2t, whh2t, bih2, bhh2, wih3t, whh3t, bih3, bhh3):
    def cell(hv, wih_ref, whh_ref, bih_ref, bhh_ref):
        gi = jnp.dot(hv, wih_ref[...], preferred_element_type=jnp.float32) + bih_ref[...]
        gh = jnp.dot(hv, whh_ref[...], preferred_element_type=jnp.float32) + bhh_ref[...]
        r = jax.nn.sigmoid(gi[:, :_H] + gh[:, :_H])
        z = jax.nn.sigmoid(gi[:, _H:2 * _H] + gh[:, _H:2 * _H])
        n = jnp.tanh(gi[:, 2 * _H:] + r * gh[:, 2 * _H:])
        return (1.0 - z) * n + z * hv

    def body(cnt_ref, hm_ref, wih2_ref, whh2_ref, bih2_ref, bhh2_ref,
             wih3_ref, whh3_ref, bih3_ref, bhh3_ref, o_ref):
        pid = pl.program_id(0)

        @pl.when(pid * _BG < cnt_ref[0])
        def _():
            hv0 = hm_ref[...]

            def it(_, hv):
                hv = cell(hv, wih2_ref, whh2_ref, bih2_ref, bhh2_ref)
                return cell(hv, wih3_ref, whh3_ref, bih3_ref, bhh3_ref)

            o_ref[...] = lax.fori_loop(0, _MAXR, it, hv0)

    wspec = lambda sh: pl.BlockSpec(sh, lambda i, cnt: (0, 0))
    grid_spec = pltpu.PrefetchScalarGridSpec(
        num_scalar_prefetch=1,
        grid=(_NBG,),
        in_specs=[
            pl.BlockSpec((_BG, _H), lambda i, cnt: (i, 0)),
            wspec((_H, 3 * _H)), wspec((_H, 3 * _H)),
            wspec((1, 3 * _H)), wspec((1, 3 * _H)),
            wspec((_H, 3 * _H)), wspec((_H, 3 * _H)),
            wspec((1, 3 * _H)), wspec((1, 3 * _H)),
        ],
        out_specs=pl.BlockSpec((_BG, _H), lambda i, cnt: (i, 0)),
    )
    return pl.pallas_call(
        body,
        grid_spec=grid_spec,
        out_shape=jax.ShapeDtypeStruct((_N, _H), jnp.float32),
    )(count, hm, wih2t, whh2t, bih2, bhh2, wih3t, whh3t, bih3, bhh3)


def _final(hf, wt, b):
    def body(h_ref, w_ref, b_ref, o_ref):
        o_ref[...] = (
            jnp.dot(h_ref[...], w_ref[...], preferred_element_type=jnp.float32)
            + b_ref[...])

    return pl.pallas_call(
        body,
        grid=(_N // _BP,),
        in_specs=[
            pl.BlockSpec((_BP, _H), lambda i: (i, 0)),
            pl.BlockSpec((_H, 2), lambda i: (0, 0)),
            pl.BlockSpec((1, 2), lambda i: (0, 0)),
        ],
        out_specs=pl.BlockSpec((_BP, 2), lambda i: (i, 0)),
        out_shape=jax.ShapeDtypeStruct((_N, 2), jnp.float32),
    )(hf, wt, b)


def kernel(x, edge_index, Wp, bp, W0, b0, W1, b1, Wt, bt,
           Wih2, Whh2, bih2, bhh2, Wih3, Whh3, bih3, bhh3, Wout, bout):
    row = edge_index[0]
    col = edge_index[1]

    wpt = Wp.T
    bp2 = bp[None, :]
    w0t = W0.T
    w1t = W1.T
    wtt = Wt.T
    bt2 = bt[None, :]
    wih2t = Wih2.T
    whh2t = Whh2.T
    bih2r = bih2[None, :]
    bhh2r = bhh2[None, :]
    wih3t = Wih3.T
    whh3t = Whh3.T
    bih3r = bih3[None, :]
    bhh3r = bhh3[None, :]
    woutt = Wout.T
    bout2 = bout[None, :]

    h = _proj(x, wpt, bp2)

    def layer(h, wlt, bl):
        parts = _edge_agg(h, row, col)
        h2, tau = _dense_layer(h, parts[0], parts[1], wlt, bl[None, :], wtt, bt2)
        mask = tau[:, 0] < _TAU
        count = jnp.sum(mask.astype(jnp.int32))[None]
        idx = jnp.nonzero(mask, size=_N, fill_value=_N)[0].astype(jnp.int32)
        idxg = jnp.minimum(idx, _N - 1)
        hm = _compact_gather(h2, idxg)
        res = _gru_recursion(hm, count, wih2t, whh2t, bih2r, bhh2r,
                             wih3t, whh3t, bih3r, bhh3r)
        return _merge_scatter(h2, res, idx)

    h1f = layer(h, w0t, b0)
    h2f = layer(h1f, w1t, b1)
    return _final(h2f, woutt, bout2)


# speedup vs baseline: 6.8994x; 2.7334x over previous
import functools

import jax
import jax.numpy as jnp
from jax import lax
from jax.experimental import pallas as pl
from jax.experimental.pallas import tpu as pltpu
from jax.experimental.pallas import tpu_sc as plsc

_N = 10000
_E = 320000
_H = 128
_MAXR = 10
_TAU = 0.005

_NW = 32
_EPW = _E // _NW
_K = 80
_NCH = _EPW // _K
_RCH = _N // _K

_PAD = 2048
_BP = 2000
_BG = 400
_NBG = _N // _BG


def _sc_mesh():
    return plsc.VectorSubcoreMesh(
        core_axis_name="c", subcore_axis_name="s", num_cores=2, num_subcores=16)


def _edge_agg(h, row, col):
    @functools.partial(
        pl.kernel,
        out_type=jax.ShapeDtypeStruct((2, _N, _H), jnp.float32),
        mesh=_sc_mesh(),
        scratch_types=[
            pltpu.VMEM((_K,), jnp.int32),
            pltpu.VMEM((_K,), jnp.int32),
            pltpu.VMEM((_K,), jnp.int32),
            pltpu.VMEM((_K,), jnp.int32),
            pltpu.VMEM((_K, _H), jnp.float32),
            pltpu.VMEM((_K, _H), jnp.float32),
            pltpu.VMEM((_K, _H), jnp.float32),
            pltpu.VMEM((_K, _H), jnp.float32),
            pltpu.VMEM_SHARED((_N, _H), jnp.float32),
            pltpu.SemaphoreType.DMA,
            pltpu.SemaphoreType.DMA,
            pltpu.SemaphoreType.DMA,
            pltpu.SemaphoreType.DMA,
        ],
    )
    def ek(h_hbm, row_hbm, col_hbm, out_hbm,
           idx_r0, idx_c0, idx_r1, idx_c1,
           buf_a0, buf_b0, buf_a1, buf_b1, acc,
           sem_a0, sem_b0, sem_a1, sem_b1):
        c = lax.axis_index("c")
        s = lax.axis_index("s")
        w = c * 16 + s

        def zero_body(i, _):
            for q in range(_H // 16):
                buf_a0[i, pl.ds(q * 16, 16)] = jnp.zeros((16,), jnp.float32)
            return 0
        lax.fori_loop(0, _K, zero_body, 0)

        for t in range(8):
            j = s + t * 16

            @pl.when(j < _RCH)
            def _():
                pltpu.sync_copy(buf_a0, acc.at[pl.ds(j * _K, _K)])

        plsc.subcore_barrier()

        def issue(ch, idx_r, idx_c, buf_a, buf_b, sem_a, sem_b):
            base = w * _EPW + ch * _K
            pltpu.sync_copy(row_hbm.at[pl.ds(base, _K)], idx_r)
            pltpu.sync_copy(col_hbm.at[pl.ds(base, _K)], idx_c)
            pltpu.async_copy(h_hbm.at[idx_r], buf_a, sem_a)
            pltpu.async_copy(h_hbm.at[idx_c], buf_b, sem_b)

        def consume(idx_r, idx_c, buf_a, buf_b, sem_a, sem_b):
            pltpu.make_async_copy(h_hbm.at[idx_r], buf_a, sem_a).wait()
            pltpu.make_async_copy(h_hbm.at[idx_c], buf_b, sem_b).wait()

            def diff_body(i, _):
                for q in range(_H // 16):
                    sl = pl.ds(q * 16, 16)
                    buf_a[i, sl] = jnp.abs(buf_a[i, sl] - buf_b[i, sl])
                return 0
            lax.fori_loop(0, _K, diff_body, 0)
            pltpu.sync_copy(buf_a, acc.at[idx_r], add=True)

        issue(0, idx_r0, idx_c0, buf_a0, buf_b0, sem_a0, sem_b0)

        def pair_body(t, _):
            ch0 = 2 * t
            issue(ch0 + 1, idx_r1, idx_c1, buf_a1, buf_b1, sem_a1, sem_b1)
            consume(idx_r0, idx_c0, buf_a0, buf_b0, sem_a0, sem_b0)
            issue(ch0 + 2, idx_r0, idx_c0, buf_a0, buf_b0, sem_a0, sem_b0)
            consume(idx_r1, idx_c1, buf_a1, buf_b1, sem_a1, sem_b1)
            return 0
        lax.fori_loop(0, (_NCH - 1) // 2, pair_body, 0)

        consume(idx_r0, idx_c0, buf_a0, buf_b0, sem_a0, sem_b0)

        plsc.subcore_barrier()

        for t in range(8):
            j = s + t * 16

            @pl.when(j < _RCH)
            def _():
                pltpu.sync_copy(acc.at[pl.ds(j * _K, _K)], buf_a0)
                pltpu.sync_copy(buf_a0, out_hbm.at[c, pl.ds(j * _K, _K)])

    return ek(h, row, col)


def _compact_gather(h2, idxg):
    @functools.partial(
        pl.kernel,
        out_type=jax.ShapeDtypeStruct((_N, _H), jnp.float32),
        mesh=_sc_mesh(),
        scratch_types=[
            pltpu.VMEM((_K,), jnp.int32),
            pltpu.VMEM((_K, _H), jnp.float32),
            pltpu.SemaphoreType.DMA,
        ],
    )
    def gk(h_hbm, idx_hbm, out_hbm, idx_v, buf, sem):
        c = lax.axis_index("c")
        s = lax.axis_index("s")
        w = c * 16 + s
        for t in range(4):
            j = w + t * _NW

            @pl.when(j < _RCH)
            def _():
                pltpu.sync_copy(idx_hbm.at[pl.ds(j * _K, _K)], idx_v)
                pltpu.async_copy(h_hbm.at[idx_v], buf, sem).wait()
                pltpu.sync_copy(buf, out_hbm.at[pl.ds(j * _K, _K)])

    return gk(h2, idxg)


def _merge_scatter(h2, res, idxs):
    @functools.partial(
        pl.kernel,
        out_type=jax.ShapeDtypeStruct((_N + _PAD, _H), jnp.float32),
        mesh=_sc_mesh(),
        scratch_types=[
            pltpu.VMEM((_K,), jnp.int32),
            pltpu.VMEM((_K, _H), jnp.float32),
            pltpu.SemaphoreType.DMA,
        ],
    )
    def mk(h_hbm, res_hbm, idx_hbm, out_hbm, idx_v, buf, sem):
        c = lax.axis_index("c")
        s = lax.axis_index("s")

        @pl.when(c == 0)
        def _():
            for t in range(8):
                j = s + t * 16

                @pl.when(j < _RCH)
                def _():
                    pltpu.sync_copy(h_hbm.at[pl.ds(j * _K, _K)], buf)
                    pltpu.sync_copy(buf, out_hbm.at[pl.ds(j * _K, _K)])

            plsc.subcore_barrier()

            for t in range(8):
                j = s + t * 16

                @pl.when(j < _RCH)
                def _():
                    pltpu.sync_copy(idx_hbm.at[pl.ds(j * _K, _K)], idx_v)
                    pltpu.sync_copy(res_hbm.at[pl.ds(j * _K, _K)], buf)
                    pltpu.sync_copy(buf, out_hbm.at[idx_v])

    return mk(h2, res, idxs)


def _proj(x, wt, b):
    def body(x_ref, w_ref, b_ref, o_ref):
        o_ref[...] = jax.nn.relu(
            jnp.dot(x_ref[...], w_ref[...], preferred_element_type=jnp.float32)
            + b_ref[...])

    return pl.pallas_call(
        body,
        grid=(_N // _BP,),
        in_specs=[
            pl.BlockSpec((_BP, _H), lambda i: (i, 0)),
            pl.BlockSpec((_H, _H), lambda i: (0, 0)),
            pl.BlockSpec((1, _H), lambda i: (0, 0)),
        ],
        out_specs=pl.BlockSpec((_BP, _H), lambda i: (i, 0)),
        out_shape=jax.ShapeDtypeStruct((_N, _H), jnp.float32),
    )(x, wt, b)


def _dense_layer(h, agg0, agg1, wlt, b, wtt, bt):
    def body(h_ref, a0_ref, a1_ref, wl_ref, b_ref, wt_ref, bt_ref,
             h2_ref, tau_ref):
        hc = jnp.concatenate([h_ref[...], a0_ref[...] + a1_ref[...]], axis=-1)
        h2 = jax.nn.relu(
            jnp.dot(hc, wl_ref[...], preferred_element_type=jnp.float32)
            + b_ref[...])
        h2_ref[...] = h2
        t = jnp.dot(h2, wt_ref[...], preferred_element_type=jnp.float32)
        tau_ref[...] = jax.nn.softplus(t + bt_ref[...])

    return pl.pallas_call(
        body,
        grid=(_N // _BP,),
        in_specs=[
            pl.BlockSpec((_BP, _H), lambda i: (i, 0)),
            pl.BlockSpec((_BP, _H), lambda i: (i, 0)),
            pl.BlockSpec((_BP, _H), lambda i: (i, 0)),
            pl.BlockSpec((2 * _H, _H), lambda i: (0, 0)),
            pl.BlockSpec((1, _H), lambda i: (0, 0)),
            pl.BlockSpec((_H, 1), lambda i: (0, 0)),
            pl.BlockSpec((1, 1), lambda i: (0, 0)),
        ],
        out_specs=[
            pl.BlockSpec((_BP, _H), lambda i: (i, 0)),
            pl.BlockSpec((_BP, 1), lambda i: (i, 0)),
        ],
        out_shape=[
            jax.ShapeDtypeStruct((_N, _H), jnp.float32),
            jax.ShapeDtypeStruct((_N, 1), jnp.float32),
        ],
    )(h, agg0, agg1, wlt, b, wtt, bt)


def _gru_recursion(hm, count, wih2t, whh2t, bih2, bhh2, wih3t, whh3t, bih3, bhh3):
    def cell(hv, wih_ref, whh_ref, bih_ref, bhh_ref):
        gi = jnp.dot(hv, wih_ref[...], preferred_element_type=jnp.float32) + bih_ref[...]
        gh = jnp.dot(hv, whh_ref[...], preferred_element_type=jnp.float32) + bhh_ref[...]
        r = jax.nn.sigmoid(gi[:, :_H] + gh[:, :_H])
        z = jax.nn.sigmoid(gi[:, _H:2 * _H] + gh[:, _H:2 * _H])
        n = jnp.tanh(gi[:, 2 * _H:] + r * gh[:, 2 * _H:])
        return (1.0 - z) * n + z * hv

    def body(cnt_ref, hm_ref, wih2_ref, whh2_ref, bih2_ref, bhh2_ref,
             wih3_ref, whh3_ref, bih3_ref, bhh3_ref, o_ref):
        pid = pl.program_id(0)

        @pl.when(pid * _BG < cnt_ref[0])
        def _():
            hv0 = hm_ref[...]

            def it(_, hv):
                hv = cell(hv, wih2_ref, whh2_ref, bih2_ref, bhh2_ref)
                return cell(hv, wih3_ref, whh3_ref, bih3_ref, bhh3_ref)

            o_ref[...] = lax.fori_loop(0, _MAXR, it, hv0)

    wspec = lambda sh: pl.BlockSpec(sh, lambda i, cnt: (0, 0))
    grid_spec = pltpu.PrefetchScalarGridSpec(
        num_scalar_prefetch=1,
        grid=(_NBG,),
        in_specs=[
            pl.BlockSpec((_BG, _H), lambda i, cnt: (i, 0)),
            wspec((_H, 3 * _H)), wspec((_H, 3 * _H)),
            wspec((1, 3 * _H)), wspec((1, 3 * _H)),
            wspec((_H, 3 * _H)), wspec((_H, 3 * _H)),
            wspec((1, 3 * _H)), wspec((1, 3 * _H)),
        ],
        out_specs=pl.BlockSpec((_BG, _H), lambda i, cnt: (i, 0)),
    )
    return pl.pallas_call(
        body,
        grid_spec=grid_spec,
        out_shape=jax.ShapeDtypeStruct((_N, _H), jnp.float32),
    )(count, hm, wih2t, whh2t, bih2, bhh2, wih3t, whh3t, bih3, bhh3)


def _final(hf, wt, b):
    def body(h_ref, w_ref, b_ref, o_ref):
        o_ref[...] = (
            jnp.dot(h_ref[...], w_ref[...], preferred_element_type=jnp.float32)
            + b_ref[...])

    return pl.pallas_call(
        body,
        grid=(_N // _BP,),
        in_specs=[
            pl.BlockSpec((_BP, _H), lambda i: (i, 0)),
            pl.BlockSpec((_H, 2), lambda i: (0, 0)),
            pl.BlockSpec((1, 2), lambda i: (0, 0)),
        ],
        out_specs=pl.BlockSpec((_BP, 2), lambda i: (i, 0)),
        out_shape=jax.ShapeDtypeStruct((_N, 2), jnp.float32),
    )(hf, wt, b)


def kernel(x, edge_index, Wp, bp, W0, b0, W1, b1, Wt, bt,
           Wih2, Whh2, bih2, bhh2, Wih3, Whh3, bih3, bhh3, Wout, bout):
    row = edge_index[0]
    col = edge_index[1]

    wpt = Wp.T
    bp2 = bp[None, :]
    w0t = W0.T
    w1t = W1.T
    wtt = Wt.T
    bt2 = bt[None, :]
    wih2t = Wih2.T
    whh2t = Whh2.T
    bih2r = bih2[None, :]
    bhh2r = bhh2[None, :]
    wih3t = Wih3.T
    whh3t = Whh3.T
    bih3r = bih3[None, :]
    bhh3r = bhh3[None, :]
    woutt = Wout.T
    bout2 = bout[None, :]

    h = _proj(x, wpt, bp2)

    def layer(h, wlt, bl):
        parts = _edge_agg(h, row, col)
        h2, tau = _dense_layer(h, parts[0], parts[1], wlt, bl[None, :], wtt, bt2)
        mask = tau[:, 0] < _TAU
        count = jnp.sum(mask.astype(jnp.int32))[None]
        idx = jnp.nonzero(mask, size=_N, fill_value=-1)[0].astype(jnp.int32)
        ar = jnp.arange(_N, dtype=jnp.int32)
        idxg = jnp.where(idx >= 0, idx, ar)
        idxs = jnp.where(idx >= 0, idx, _N + (ar % _PAD))
        hm = _compact_gather(h2, idxg)
        res = _gru_recursion(hm, count, wih2t, whh2t, bih2r, bhh2r,
                             wih3t, whh3t, bih3r, bhh3r)
        return _merge_scatter(h2, res, idxs)

    h1f = layer(h, w0t, b0)
    h2f = layer(h1f, w1t, b1)
    return _final(h2f, woutt, bout2)


# speedup vs baseline: 7.8538x; 1.1383x over previous
import functools

import jax
import jax.numpy as jnp
from jax import lax
from jax.experimental import pallas as pl
from jax.experimental.pallas import tpu as pltpu
from jax.experimental.pallas import tpu_sc as plsc

_N = 10000
_E = 320000
_H = 128
_MAXR = 10
_TAU = 0.005

_NW = 32
_EPW = _E // _NW
_K = 80
_NCH = _EPW // _K
_RCH = _N // _K

_PAD = 2048
_BP = 2000
_BG = 400
_NBG = _N // _BG


def _sc_mesh():
    return plsc.VectorSubcoreMesh(
        core_axis_name="c", subcore_axis_name="s", num_cores=2, num_subcores=16)


def _edge_agg(h, row, col):
    @functools.partial(
        pl.kernel,
        out_type=jax.ShapeDtypeStruct((2, _N, _H), jnp.float32),
        mesh=_sc_mesh(),
        scratch_types=[
            pltpu.VMEM((_K,), jnp.int32),
            pltpu.VMEM((_K,), jnp.int32),
            pltpu.VMEM((_K,), jnp.int32),
            pltpu.VMEM((_K,), jnp.int32),
            pltpu.VMEM((_K, _H), jnp.float32),
            pltpu.VMEM((_K, _H), jnp.float32),
            pltpu.VMEM((_K, _H), jnp.float32),
            pltpu.VMEM((_K, _H), jnp.float32),
            pltpu.VMEM_SHARED((_N, _H), jnp.float32),
            pltpu.SemaphoreType.DMA,
            pltpu.SemaphoreType.DMA,
            pltpu.SemaphoreType.DMA,
            pltpu.SemaphoreType.DMA,
            pltpu.SemaphoreType.DMA,
            pltpu.SemaphoreType.DMA,
            pltpu.SemaphoreType.DMA,
            pltpu.SemaphoreType.DMA,
        ],
    )
    def ek(h_hbm, row_hbm, col_hbm, out_hbm,
           idx_r0, idx_c0, idx_r1, idx_c1,
           buf_a0, buf_b0, buf_a1, buf_b1, acc,
           sem_a0, sem_b0, sem_a1, sem_b1,
           sem_i0, sem_j0, sem_i1, sem_j1):
        c = lax.axis_index("c")
        s = lax.axis_index("s")
        w = c * 16 + s

        def zero_body(i, _):
            for q in range(_H // 16):
                buf_a0[i, pl.ds(q * 16, 16)] = jnp.zeros((16,), jnp.float32)
            return 0
        lax.fori_loop(0, _K, zero_body, 0)

        for t in range(8):
            j = s + t * 16

            @pl.when(j < _RCH)
            def _():
                pltpu.sync_copy(buf_a0, acc.at[pl.ds(j * _K, _K)])

        plsc.subcore_barrier()

        def idx_copy(ch, idx_r, idx_c, sem_ir, sem_ic):
            base = w * _EPW + ch * _K
            pltpu.async_copy(row_hbm.at[pl.ds(base, _K)], idx_r, sem_ir)
            pltpu.async_copy(col_hbm.at[pl.ds(base, _K)], idx_c, sem_ic)

        def idx_wait(idx_r, idx_c, sem_ir, sem_ic):
            pltpu.make_async_copy(row_hbm.at[pl.ds(0, _K)], idx_r, sem_ir).wait()
            pltpu.make_async_copy(col_hbm.at[pl.ds(0, _K)], idx_c, sem_ic).wait()

        def gathers(idx_r, idx_c, buf_a, buf_b, sem_a, sem_b):
            pltpu.async_copy(h_hbm.at[idx_r], buf_a, sem_a)
            pltpu.async_copy(h_hbm.at[idx_c], buf_b, sem_b)

        def consume(idx_r, idx_c, buf_a, buf_b, sem_a, sem_b):
            pltpu.make_async_copy(h_hbm.at[idx_r], buf_a, sem_a).wait()
            pltpu.make_async_copy(h_hbm.at[idx_c], buf_b, sem_b).wait()

            def diff_body(i, _):
                for q in range(_H // 16):
                    sl = pl.ds(q * 16, 16)
                    buf_a[i, sl] = jnp.abs(buf_a[i, sl] - buf_b[i, sl])
                return 0
            lax.fori_loop(0, _K, diff_body, 0)
            pltpu.sync_copy(buf_a, acc.at[idx_r], add=True)

        set0 = (idx_r0, idx_c0, buf_a0, buf_b0, sem_a0, sem_b0, sem_i0, sem_j0)
        set1 = (idx_r1, idx_c1, buf_a1, buf_b1, sem_a1, sem_b1, sem_i1, sem_j1)

        def half(chg, cur, nxt):
            ir, ic, ba, bb, sa, sb, si, sj = cur
            nir, nic, nba, nbb, nsa, nsb, nsi, nsj = nxt
            idx_wait(nir, nic, nsi, nsj)
            gathers(nir, nic, nba, nbb, nsa, nsb)
            consume(ir, ic, ba, bb, sa, sb)

            @pl.when(chg + 2 < _NCH)
            def _():
                idx_copy(chg + 2, ir, ic, si, sj)

        idx_copy(0, idx_r0, idx_c0, sem_i0, sem_j0)
        idx_wait(idx_r0, idx_c0, sem_i0, sem_j0)
        gathers(idx_r0, idx_c0, buf_a0, buf_b0, sem_a0, sem_b0)
        idx_copy(1, idx_r1, idx_c1, sem_i1, sem_j1)

        def pair_body(t, _):
            half(2 * t, set0, set1)
            half(2 * t + 1, set1, set0)
            return 0
        lax.fori_loop(0, (_NCH - 1) // 2, pair_body, 0)

        consume(idx_r0, idx_c0, buf_a0, buf_b0, sem_a0, sem_b0)

        plsc.subcore_barrier()

        for t in range(8):
            j = s + t * 16

            @pl.when(j < _RCH)
            def _():
                pltpu.sync_copy(acc.at[pl.ds(j * _K, _K)], buf_a0)
                pltpu.sync_copy(buf_a0, out_hbm.at[c, pl.ds(j * _K, _K)])

    return ek(h, row, col)


def _compact_gather(h2, idxg):
    @functools.partial(
        pl.kernel,
        out_type=jax.ShapeDtypeStruct((_N, _H), jnp.float32),
        mesh=_sc_mesh(),
        scratch_types=[
            pltpu.VMEM((_K,), jnp.int32),
            pltpu.VMEM((_K, _H), jnp.float32),
            pltpu.SemaphoreType.DMA,
        ],
    )
    def gk(h_hbm, idx_hbm, out_hbm, idx_v, buf, sem):
        c = lax.axis_index("c")
        s = lax.axis_index("s")
        w = c * 16 + s
        for t in range(4):
            j = w + t * _NW

            @pl.when(j < _RCH)
            def _():
                pltpu.sync_copy(idx_hbm.at[pl.ds(j * _K, _K)], idx_v)
                pltpu.async_copy(h_hbm.at[idx_v], buf, sem).wait()
                pltpu.sync_copy(buf, out_hbm.at[pl.ds(j * _K, _K)])

    return gk(h2, idxg)


def _merge_scatter(h2, res, idxs):
    @functools.partial(
        pl.kernel,
        out_type=jax.ShapeDtypeStruct((_N + _PAD, _H), jnp.float32),
        mesh=_sc_mesh(),
        scratch_types=[
            pltpu.VMEM((_K,), jnp.int32),
            pltpu.VMEM((_K, _H), jnp.float32),
            pltpu.SemaphoreType.DMA,
        ],
    )
    def mk(h_hbm, res_hbm, idx_hbm, out_hbm, idx_v, buf, sem):
        c = lax.axis_index("c")
        s = lax.axis_index("s")

        @pl.when(c == 0)
        def _():
            for t in range(8):
                j = s + t * 16

                @pl.when(j < _RCH)
                def _():
                    pltpu.sync_copy(h_hbm.at[pl.ds(j * _K, _K)], buf)
                    pltpu.sync_copy(buf, out_hbm.at[pl.ds(j * _K, _K)])

            plsc.subcore_barrier()

            for t in range(8):
                j = s + t * 16

                @pl.when(j < _RCH)
                def _():
                    pltpu.sync_copy(idx_hbm.at[pl.ds(j * _K, _K)], idx_v)
                    pltpu.sync_copy(res_hbm.at[pl.ds(j * _K, _K)], buf)
                    pltpu.sync_copy(buf, out_hbm.at[idx_v])

    return mk(h2, res, idxs)


def _proj(x, wt, b):
    def body(x_ref, w_ref, b_ref, o_ref):
        o_ref[...] = jax.nn.relu(
            jnp.dot(x_ref[...], w_ref[...], preferred_element_type=jnp.float32)
            + b_ref[...])

    return pl.pallas_call(
        body,
        grid=(_N // _BP,),
        in_specs=[
            pl.BlockSpec((_BP, _H), lambda i: (i, 0)),
            pl.BlockSpec((_H, _H), lambda i: (0, 0)),
            pl.BlockSpec((1, _H), lambda i: (0, 0)),
        ],
        out_specs=pl.BlockSpec((_BP, _H), lambda i: (i, 0)),
        out_shape=jax.ShapeDtypeStruct((_N, _H), jnp.float32),
    )(x, wt, b)


def _dense_layer(h, agg0, agg1, wlt, b, wtt, bt):
    def body(h_ref, a0_ref, a1_ref, wl_ref, b_ref, wt_ref, bt_ref,
             h2_ref, tau_ref):
        hc = jnp.concatenate([h_ref[...], a0_ref[...] + a1_ref[...]], axis=-1)
        h2 = jax.nn.relu(
            jnp.dot(hc, wl_ref[...], preferred_element_type=jnp.float32)
            + b_ref[...])
        h2_ref[...] = h2
        t = jnp.dot(h2, wt_ref[...], preferred_element_type=jnp.float32)
        tau_ref[...] = jax.nn.softplus(t + bt_ref[...])

    return pl.pallas_call(
        body,
        grid=(_N // _BP,),
        in_specs=[
            pl.BlockSpec((_BP, _H), lambda i: (i, 0)),
            pl.BlockSpec((_BP, _H), lambda i: (i, 0)),
            pl.BlockSpec((_BP, _H), lambda i: (i, 0)),
            pl.BlockSpec((2 * _H, _H), lambda i: (0, 0)),
            pl.BlockSpec((1, _H), lambda i: (0, 0)),
            pl.BlockSpec((_H, 1), lambda i: (0, 0)),
            pl.BlockSpec((1, 1), lambda i: (0, 0)),
        ],
        out_specs=[
            pl.BlockSpec((_BP, _H), lambda i: (i, 0)),
            pl.BlockSpec((_BP, 1), lambda i: (i, 0)),
        ],
        out_shape=[
            jax.ShapeDtypeStruct((_N, _H), jnp.float32),
            jax.ShapeDtypeStruct((_N, 1), jnp.float32),
        ],
    )(h, agg0, agg1, wlt, b, wtt, bt)


def _gru_recursion(hm, count, wih2t, whh2t, bih2, bhh2, wih3t, whh3t, bih3, bhh3):
    def cell(hv, wih_ref, whh_ref, bih_ref, bhh_ref):
        gi = jnp.dot(hv, wih_ref[...], preferred_element_type=jnp.float32) + bih_ref[...]
        gh = jnp.dot(hv, whh_ref[...], preferred_element_type=jnp.float32) + bhh_ref[...]
        r = jax.nn.sigmoid(gi[:, :_H] + gh[:, :_H])
        z = jax.nn.sigmoid(gi[:, _H:2 * _H] + gh[:, _H:2 * _H])
        n = jnp.tanh(gi[:, 2 * _H:] + r * gh[:, 2 * _H:])
        return (1.0 - z) * n + z * hv

    def body(cnt_ref, hm_ref, wih2_ref, whh2_ref, bih2_ref, bhh2_ref,
             wih3_ref, whh3_ref, bih3_ref, bhh3_ref, o_ref):
        pid = pl.program_id(0)

        @pl.when(pid * _BG < cnt_ref[0])
        def _():
            hv0 = hm_ref[...]

            def it(_, hv):
                hv = cell(hv, wih2_ref, whh2_ref, bih2_ref, bhh2_ref)
                return cell(hv, wih3_ref, whh3_ref, bih3_ref, bhh3_ref)

            o_ref[...] = lax.fori_loop(0, _MAXR, it, hv0)

    wspec = lambda sh: pl.BlockSpec(sh, lambda i, cnt: (0, 0))
    grid_spec = pltpu.PrefetchScalarGridSpec(
        num_scalar_prefetch=1,
        grid=(_NBG,),
        in_specs=[
            pl.BlockSpec((_BG, _H), lambda i, cnt: (i, 0)),
            wspec((_H, 3 * _H)), wspec((_H, 3 * _H)),
            wspec((1, 3 * _H)), wspec((1, 3 * _H)),
            wspec((_H, 3 * _H)), wspec((_H, 3 * _H)),
            wspec((1, 3 * _H)), wspec((1, 3 * _H)),
        ],
        out_specs=pl.BlockSpec((_BG, _H), lambda i, cnt: (i, 0)),
    )
    return pl.pallas_call(
        body,
        grid_spec=grid_spec,
        out_shape=jax.ShapeDtypeStruct((_N, _H), jnp.float32),
    )(count, hm, wih2t, whh2t, bih2, bhh2, wih3t, whh3t, bih3, bhh3)


def _final(hf, wt, b):
    def body(h_ref, w_ref, b_ref, o_ref):
        o_ref[...] = (
            jnp.dot(h_ref[...], w_ref[...], preferred_element_type=jnp.float32)
            + b_ref[...])

    return pl.pallas_call(
        body,
        grid=(_N // _BP,),
        in_specs=[
            pl.BlockSpec((_BP, _H), lambda i: (i, 0)),
            pl.BlockSpec((_H, 2), lambda i: (0, 0)),
            pl.BlockSpec((1, 2), lambda i: (0, 0)),
        ],
        out_specs=pl.BlockSpec((_BP, 2), lambda i: (i, 0)),
        out_shape=jax.ShapeDtypeStruct((_N, 2), jnp.float32),
    )(hf, wt, b)


def kernel(x, edge_index, Wp, bp, W0, b0, W1, b1, Wt, bt,
           Wih2, Whh2, bih2, bhh2, Wih3, Whh3, bih3, bhh3, Wout, bout):
    row = edge_index[0]
    col = edge_index[1]

    wpt = Wp.T
    bp2 = bp[None, :]
    w0t = W0.T
    w1t = W1.T
    wtt = Wt.T
    bt2 = bt[None, :]
    wih2t = Wih2.T
    whh2t = Whh2.T
    bih2r = bih2[None, :]
    bhh2r = bhh2[None, :]
    wih3t = Wih3.T
    whh3t = Whh3.T
    bih3r = bih3[None, :]
    bhh3r = bhh3[None, :]
    woutt = Wout.T
    bout2 = bout[None, :]

    h = _proj(x, wpt, bp2)

    def layer(h, wlt, bl):
        parts = _edge_agg(h, row, col)
        h2, tau = _dense_layer(h, parts[0], parts[1], wlt, bl[None, :], wtt, bt2)
        mask = tau[:, 0] < _TAU
        count = jnp.sum(mask.astype(jnp.int32))[None]
        idx = jnp.nonzero(mask, size=_N, fill_value=-1)[0].astype(jnp.int32)
        ar = jnp.arange(_N, dtype=jnp.int32)
        idxg = jnp.where(idx >= 0, idx, ar)
        idxs = jnp.where(idx >= 0, idx, _N + (ar % _PAD))
        hm = _compact_gather(h2, idxg)
        res = _gru_recursion(hm, count, wih2t, whh2t, bih2r, bhh2r,
                             wih3t, whh3t, bih3r, bhh3r)
        return _merge_scatter(h2, res, idxs)

    h1f = layer(h, w0t, b0)
    h2f = layer(h1f, w1t, b1)
    return _final(h2f, woutt, bout2)
